# baseline (speedup 1.0000x reference)
import functools

import jax
import jax.numpy as jnp
from jax import lax
from jax.experimental import pallas as pl
from jax.experimental.pallas import tpu as pltpu
from jax.experimental.pallas import tpu_sc as plsc

B = 16384
D = 64
L = 16
NC, NS = 2, 16
NW = NC * NS
BPW = B // NW
NCHUNK = 4
CHUNK = BPW // NCHUNK

_mesh = plsc.VectorSubcoreMesh(core_axis_name="c", subcore_axis_name="s")


@functools.partial(
    pl.kernel,
    out_type=(
        jax.ShapeDtypeStruct((B, L), jnp.float32),
        jax.ShapeDtypeStruct((NW, L), jnp.float32),
    ),
    mesh=_mesh,
    compiler_params=pltpu.CompilerParams(use_tc_tiling_on_sc=False),
    scratch_types=[
        pltpu.VMEM((NCHUNK, CHUNK), jnp.int32),
        pltpu.VMEM((NCHUNK, CHUNK), jnp.int32),
        pltpu.VMEM((NCHUNK, CHUNK), jnp.int32),
        pltpu.VMEM((BPW, D), jnp.float32),
        pltpu.VMEM((BPW, D), jnp.float32),
        pltpu.VMEM((BPW, D), jnp.float32),
        pltpu.VMEM((BPW, L), jnp.float32),
        pltpu.VMEM((L,), jnp.float32),
        pltpu.SemaphoreType.DMA,
    ],
)
def _bpr_sc(uid_hbm, pid_hbm, nid_hbm, w_hbm, h_hbm, d_hbm, reg_hbm,
            uid_v, pid_v, nid_v, u_v, p_v, n_v, d_v, racc_v, sem):
    wid = lax.axis_index("s") * NC + lax.axis_index("c")
    base = wid * NCHUNK
    pltpu.sync_copy(uid_hbm.at[pl.ds(base, NCHUNK)], uid_v)
    pltpu.sync_copy(pid_hbm.at[pl.ds(base, NCHUNK)], pid_v)
    pltpu.sync_copy(nid_hbm.at[pl.ds(base, NCHUNK)], nid_v)

    copies = []
    for c in range(NCHUNK):
        sl = pl.ds(c * CHUNK, CHUNK)
        copies.append(pltpu.async_copy(w_hbm.at[uid_v.at[c]], u_v.at[sl], sem))
        copies.append(pltpu.async_copy(h_hbm.at[pid_v.at[c]], p_v.at[sl], sem))
        copies.append(pltpu.async_copy(h_hbm.at[nid_v.at[c]], n_v.at[sl], sem))
    for cp in copies:
        cp.wait()

    racc_v[...] = jnp.zeros((L,), jnp.float32)

    @pl.loop(0, BPW)
    def _(r):
        dv = None
        rs = None
        for c in range(D // L):
            sl = pl.ds(c * L, L)
            u = u_v[r, sl]
            p = p_v[r, sl]
            n = n_v[r, sl]
            contrib = u * (p - n)
            sq = u * u + p * p + n * n
            dv = contrib if dv is None else dv + contrib
            rs = sq if rs is None else rs + sq
        d_v[r, :] = dv
        racc_v[...] = racc_v[...] + rs

    pltpu.sync_copy(d_v, d_hbm.at[pl.ds(wid * BPW, BPW)])
    pltpu.sync_copy(racc_v, reg_hbm.at[wid])


def _finish_body(d_ref, reg_ref, o_ref):
    s = jnp.sum(d_ref[...], axis=1, keepdims=True)
    bpr = -jnp.sum(jax.nn.log_sigmoid(s))
    reg = 0.01 * jnp.sum(reg_ref[...])
    o_ref[...] = jnp.reshape(bpr + reg, (1, 1))


_finish = pl.pallas_call(
    _finish_body,
    out_shape=jax.ShapeDtypeStruct((1, 1), jnp.float32),
)


def kernel(data, W, H):
    uid = data[:, 0].reshape(NW * NCHUNK, CHUNK)
    pid = data[:, 1].reshape(NW * NCHUNK, CHUNK)
    nid = data[:, 2].reshape(NW * NCHUNK, CHUNK)
    d_part, reg_part = _bpr_sc(uid, pid, nid, W, H)
    return _finish(d_part, reg_part)[0, 0]


# speedup vs baseline: 1.5651x; 1.5651x over previous
import functools

import jax
import jax.numpy as jnp
from jax import lax
from jax.experimental import pallas as pl
from jax.experimental.pallas import tpu as pltpu
from jax.experimental.pallas import tpu_sc as plsc

B = 16384
D = 64
L = 16
NC, NS = 2, 16
NW = NC * NS
BPW = B // NW
C = 64
NCH = BPW // C

_mesh = plsc.VectorSubcoreMesh(core_axis_name="c", subcore_axis_name="s")


@functools.partial(
    pl.kernel,
    out_type=(
        jax.ShapeDtypeStruct((B, L), jnp.float32),
        jax.ShapeDtypeStruct((NW, L), jnp.float32),
    ),
    mesh=_mesh,
    scratch_types=[
        pltpu.VMEM((BPW,), jnp.int32),
        pltpu.VMEM((BPW,), jnp.int32),
        pltpu.VMEM((BPW,), jnp.int32),
        pltpu.VMEM((2, C, D), jnp.float32),
        pltpu.VMEM((2, C, D), jnp.float32),
        pltpu.VMEM((2, C, D), jnp.float32),
        pltpu.VMEM((BPW, L), jnp.float32),
        pltpu.VMEM((L,), jnp.float32),
        pltpu.SemaphoreType.DMA,
        pltpu.SemaphoreType.DMA,
    ],
)
def _bpr_sc(uid_hbm, pid_hbm, nid_hbm, w_hbm, h_hbm, d_hbm, reg_hbm,
            uid_v, pid_v, nid_v, u_v, p_v, n_v, d_v, racc_v, sem0, sem1):
    wid = lax.axis_index("s") * NC + lax.axis_index("c")
    pltpu.sync_copy(uid_hbm.at[wid], uid_v)
    pltpu.sync_copy(pid_hbm.at[wid], pid_v)
    pltpu.sync_copy(nid_hbm.at[wid], nid_v)

    def fire(k, buf, sem):
        @pl.loop(0, C, step=16)
        def _(j0):
            uvec = uid_v[pl.ds(k * C + j0, 16)]
            pvec = pid_v[pl.ds(k * C + j0, 16)]
            nvec = nid_v[pl.ds(k * C + j0, 16)]
            for j in range(16):
                dst = pl.ds(j0 + j, 1)
                pltpu.async_copy(w_hbm.at[pl.ds(uvec[j], 1)], u_v.at[buf].at[dst], sem)
                pltpu.async_copy(h_hbm.at[pl.ds(pvec[j], 1)], p_v.at[buf].at[dst], sem)
                pltpu.async_copy(h_hbm.at[pl.ds(nvec[j], 1)], n_v.at[buf].at[dst], sem)

    def drain(buf, sem):
        pltpu.make_async_copy(w_hbm.at[pl.ds(0, C)], u_v.at[buf], sem).wait()
        pltpu.make_async_copy(h_hbm.at[pl.ds(0, C)], p_v.at[buf], sem).wait()
        pltpu.make_async_copy(h_hbm.at[pl.ds(0, C)], n_v.at[buf], sem).wait()

    def compute(k, buf):
        @pl.loop(0, C)
        def _(j):
            dv = None
            rs = None
            for c in range(D // L):
                sl = pl.ds(c * L, L)
                u = u_v[buf, j, sl]
                p = p_v[buf, j, sl]
                n = n_v[buf, j, sl]
                contrib = u * (p - n)
                sq = u * u + p * p + n * n
                dv = contrib if dv is None else dv + contrib
                rs = sq if rs is None else rs + sq
            d_v[k * C + j, :] = dv
            racc_v[...] = racc_v[...] + rs

    racc_v[...] = jnp.zeros((L,), jnp.float32)
    fire(0, 0, sem0)
    fire(1, 1, sem1)

    @pl.loop(0, NCH, step=2)
    def _(k):
        drain(0, sem0)
        compute(k, 0)

        @pl.when(k + 2 < NCH)
        def _():
            fire(k + 2, 0, sem0)

        drain(1, sem1)
        compute(k + 1, 1)

        @pl.when(k + 3 < NCH)
        def _():
            fire(k + 3, 1, sem1)

    pltpu.sync_copy(d_v, d_hbm.at[pl.ds(wid * BPW, BPW)])
    pltpu.sync_copy(racc_v, reg_hbm.at[wid])


def _finish_body(d_ref, reg_ref, o_ref):
    s = jnp.sum(d_ref[...], axis=1, keepdims=True)
    bpr = -jnp.sum(jax.nn.log_sigmoid(s))
    reg = 0.01 * jnp.sum(reg_ref[...])
    o_ref[...] = jnp.reshape(bpr + reg, (1, 1))


_finish = pl.pallas_call(
    _finish_body,
    out_shape=jax.ShapeDtypeStruct((1, 1), jnp.float32),
)


def kernel(data, W, H):
    uid = data[:, 0].reshape(NW, BPW)
    pid = data[:, 1].reshape(NW, BPW)
    nid = data[:, 2].reshape(NW, BPW)
    d_part, reg_part = _bpr_sc(uid, pid, nid, W, H)
    return _finish(d_part, reg_part)[0, 0]


# speedup vs baseline: 1.5708x; 1.0036x over previous
import functools

import jax
import jax.numpy as jnp
from jax import lax
from jax.experimental import pallas as pl
from jax.experimental.pallas import tpu as pltpu
from jax.experimental.pallas import tpu_sc as plsc

B = 16384
D = 64
L = 16
NC, NS = 2, 16
NW = NC * NS
BPW = B // NW
C = 64
NCH = BPW // C
NSEM = 8

_mesh = plsc.VectorSubcoreMesh(core_axis_name="c", subcore_axis_name="s")


@functools.partial(
    pl.kernel,
    out_type=(
        jax.ShapeDtypeStruct((B, L), jnp.float32),
        jax.ShapeDtypeStruct((NW, L), jnp.float32),
    ),
    mesh=_mesh,
    scratch_types=[
        pltpu.VMEM((BPW,), jnp.int32),
        pltpu.VMEM((BPW,), jnp.int32),
        pltpu.VMEM((BPW,), jnp.int32),
        pltpu.VMEM((2, C, D), jnp.float32),
        pltpu.VMEM((2, C, D), jnp.float32),
        pltpu.VMEM((2, C, D), jnp.float32),
        pltpu.VMEM((BPW, L), jnp.float32),
        pltpu.VMEM((L,), jnp.float32),
        [pltpu.SemaphoreType.DMA] * NSEM,
        [pltpu.SemaphoreType.DMA] * NSEM,
    ],
)
def _bpr_sc(uid_hbm, pid_hbm, nid_hbm, w_hbm, h_hbm, d_hbm, reg_hbm,
            uid_v, pid_v, nid_v, u_v, p_v, n_v, d_v, racc_v, sems0, sems1):
    wid = lax.axis_index("s") * NC + lax.axis_index("c")
    pltpu.sync_copy(uid_hbm.at[wid], uid_v)
    pltpu.sync_copy(pid_hbm.at[wid], pid_v)
    pltpu.sync_copy(nid_hbm.at[wid], nid_v)

    def fire(k, buf, sems):
        @pl.loop(0, C, step=16)
        def _(j0):
            uvec = uid_v[pl.ds(k * C + j0, 16)]
            pvec = pid_v[pl.ds(k * C + j0, 16)]
            nvec = nid_v[pl.ds(k * C + j0, 16)]
            for j in range(16):
                dst = pl.ds(j0 + j, 1)
                sem = sems[j % NSEM]
                pltpu.async_copy(w_hbm.at[pl.ds(uvec[j], 1)], u_v.at[buf].at[dst], sem)
                pltpu.async_copy(h_hbm.at[pl.ds(pvec[j], 1)], p_v.at[buf].at[dst], sem)
                pltpu.async_copy(h_hbm.at[pl.ds(nvec[j], 1)], n_v.at[buf].at[dst], sem)

    def drain(buf, sems):
        for sem in sems:
            pltpu.make_async_copy(w_hbm.at[pl.ds(0, 3 * C // NSEM)],
                                  u_v.at[buf].at[pl.ds(0, 3 * C // NSEM)],
                                  sem).wait()

    def compute(k, buf):
        @pl.loop(0, C)
        def _(j):
            dv = None
            rs = None
            for c in range(D // L):
                sl = pl.ds(c * L, L)
                u = u_v[buf, j, sl]
                p = p_v[buf, j, sl]
                n = n_v[buf, j, sl]
                contrib = u * (p - n)
                sq = u * u + p * p + n * n
                dv = contrib if dv is None else dv + contrib
                rs = sq if rs is None else rs + sq
            d_v[k * C + j, :] = dv
            racc_v[...] = racc_v[...] + rs

    racc_v[...] = jnp.zeros((L,), jnp.float32)
    fire(0, 0, sems0)
    fire(1, 1, sems1)

    @pl.loop(0, NCH, step=2)
    def _(k):
        drain(0, sems0)
        compute(k, 0)

        @pl.when(k + 2 < NCH)
        def _():
            fire(k + 2, 0, sems0)

        drain(1, sems1)
        compute(k + 1, 1)

        @pl.when(k + 3 < NCH)
        def _():
            fire(k + 3, 1, sems1)

    pltpu.sync_copy(d_v, d_hbm.at[pl.ds(wid * BPW, BPW)])
    pltpu.sync_copy(racc_v, reg_hbm.at[wid])


def _finish_body(d_ref, reg_ref, o_ref):
    s = jnp.sum(d_ref[...], axis=1, keepdims=True)
    bpr = -jnp.sum(jax.nn.log_sigmoid(s))
    reg = 0.01 * jnp.sum(reg_ref[...])
    o_ref[...] = jnp.reshape(bpr + reg, (1, 1))


_finish = pl.pallas_call(
    _finish_body,
    out_shape=jax.ShapeDtypeStruct((1, 1), jnp.float32),
)


def kernel(data, W, H):
    uid = data[:, 0].reshape(NW, BPW)
    pid = data[:, 1].reshape(NW, BPW)
    nid = data[:, 2].reshape(NW, BPW)
    d_part, reg_part = _bpr_sc(uid, pid, nid, W, H)
    return _finish(d_part, reg_part)[0, 0]
